# baseline (speedup 1.0000x reference)
import functools

import jax
import jax.numpy as jnp
from jax import lax
from jax.experimental import pallas as pl
from jax.experimental.pallas import tpu as pltpu
from jax.experimental.pallas import tpu_sc as plsc

_L = 16
_CH = 128


def _make_kernel(B, H, V2, D, num_cores, num_subcores):
    NW = num_cores * num_subcores
    BW = B // NW
    CH, L = _CH, _L
    NQ = BW // CH
    D2 = 2 * D
    assert NQ == 4 and BW == 512

    mesh = plsc.VectorSubcoreMesh(core_axis_name="c", subcore_axis_name="s")

    scratch = [
        pltpu.VMEM((H, BW), jnp.int32),
        pltpu.VMEM((4, CH), jnp.int32),
        pltpu.VMEM((4, CH, D2), jnp.float32),
        pltpu.VMEM((2, D, CH), jnp.float32),
        pltpu.SemaphoreType.DMA,
        pltpu.SemaphoreType.DMA,
        pltpu.SemaphoreType.DMA,
        pltpu.SemaphoreType.DMA,
        pltpu.SemaphoreType.DMA,
        pltpu.SemaphoreType.DMA,
    ]

    @functools.partial(
        pl.kernel,
        mesh=mesh,
        out_type=jax.ShapeDtypeStruct((H * D, B), jnp.float32),
        scratch_types=scratch,
        compiler_params=pltpu.CompilerParams(needs_layout_passes=False),
    )
    def body(idx_hbm, table_hbm, out_hbm, idx_v, phs, bufs, obs, *sems):
        gsem = sems[:4]
        wsem = sems[4:]
        wid = lax.axis_index("s") * num_cores + lax.axis_index("c")
        col0 = lax.mul(wid, BW)

        pltpu.sync_copy(idx_hbm.at[:, pl.ds(col0, BW)], idx_v)

        iota = lax.iota(jnp.int32, L)

        def fire_gather(h, q, s):
            for r in range(CH // L):
                sl = pl.ds(CH * q + r * L, L)
                phs[s, pl.ds(r * L, L)] = lax.shift_right_logical(
                    idx_v[h, sl], 1
                )
            pltpu.async_copy(table_hbm.at[phs.at[s]], bufs.at[s], gsem[s])

        def gather_wait(s):
            pltpu.make_async_copy(
                table_hbm.at[phs.at[s]], bufs.at[s], gsem[s]
            ).wait()

        def select(h, q, s, o):
            buf = bufs.at[s]
            ob = obs.at[o]

            def grp(r, carry):
                jv = lax.add(lax.mul(r, L), iota)
                tokv = idx_v[
                    h, pl.ds(lax.add(lax.mul(CH, q), lax.mul(r, L)), L)
                ]
                hb = lax.mul(lax.bitwise_and(tokv, 1), D)
                for dblk in range(D // L):
                    vs = [
                        plsc.load_gather(buf, [jv, lax.add(hb, dblk * L + k)])
                        for k in range(L)
                    ]
                    for k in range(L):
                        ob[dblk * L + k, pl.ds(lax.mul(r, L), L)] = vs[k]
                return carry

            lax.fori_loop(0, CH // L, grp, 0)

        def out_slice(h, q):
            return out_hbm.at[
                pl.ds(lax.mul(h, D), D), pl.ds(col0 + CH * q, CH)
            ]

        def wb_fire(h, q, o):
            pltpu.async_copy(obs.at[o], out_slice(h, q), wsem[o])

        def wb_wait(h, q, o):
            pltpu.make_async_copy(obs.at[o], out_slice(h, q), wsem[o]).wait()

        def step(h, q, first, last):
            o = q % 2
            if not last:
                if q < 2:
                    fire_gather(h, q + 2, q + 2)
                else:
                    fire_gather(h + 1, q - 2, q - 2)
            gather_wait(q)
            if not first:
                if q < 2:
                    wb_wait(h - 1, q + 2, o)
                else:
                    wb_wait(h, q - 2, o)
            select(h, q, q, o)
            wb_fire(h, q, o)

        fire_gather(0, 0, 0)
        fire_gather(0, 1, 1)
        step(0, 0, first=True, last=False)
        step(0, 1, first=True, last=False)
        step(0, 2, first=False, last=False)
        step(0, 3, first=False, last=False)

        def hbody(h, carry):
            for q in range(4):
                step(h, q, first=False, last=False)
            return carry

        lax.fori_loop(1, H - 1, hbody, 0)

        step(H - 1, 0, first=False, last=False)
        step(H - 1, 1, first=False, last=False)
        step(H - 1, 2, first=False, last=True)
        step(H - 1, 3, first=False, last=True)
        wb_wait(H - 1, 2, 0)
        wb_wait(H - 1, 3, 1)

    return body


def kernel(token_ids, embeddings):
    B, H = token_ids.shape
    V, D = embeddings.shape
    info = plsc.get_sparse_core_info()
    ids_t = token_ids.T.astype(jnp.int32)
    table2 = embeddings.reshape(V // 2, 2 * D)
    out_t = _make_kernel(B, H, V // 2, D, info.num_cores, info.num_subcores)(
        ids_t, table2
    )
    return out_t.reshape(H, D, B).transpose(2, 0, 1)


# speedup vs baseline: 1.6079x; 1.6079x over previous
import functools

import jax
import jax.numpy as jnp
from jax import lax
from jax.experimental import pallas as pl
from jax.experimental.pallas import tpu as pltpu
from jax.experimental.pallas import tpu_sc as plsc

_L = 16
_CH = 128


def _make_kernel(B, H, V2, D, num_cores, num_subcores):
    NW = num_cores * num_subcores
    BW = B // NW
    CH, L = _CH, _L
    NQ = BW // CH
    D2 = 2 * D
    assert NQ == 4 and BW == 512

    mesh = plsc.VectorSubcoreMesh(core_axis_name="c", subcore_axis_name="s")

    scratch = [
        pltpu.VMEM((H, BW), jnp.int32),
        pltpu.VMEM((4, CH), jnp.int32),
        pltpu.VMEM((4, CH, D2), jnp.float32),
        pltpu.VMEM((2, D, CH), jnp.float32),
        pltpu.SemaphoreType.DMA,
        pltpu.SemaphoreType.DMA,
        pltpu.SemaphoreType.DMA,
        pltpu.SemaphoreType.DMA,
        pltpu.SemaphoreType.DMA,
        pltpu.SemaphoreType.DMA,
    ]

    @functools.partial(
        pl.kernel,
        mesh=mesh,
        out_type=jax.ShapeDtypeStruct((H * D, B), jnp.float32),
        scratch_types=scratch,
        compiler_params=pltpu.CompilerParams(needs_layout_passes=False),
    )
    def body(idx_hbm, table_hbm, out_hbm, idx_v, phs, bufs, obs, *sems):
        gsem = sems[:4]
        wsem = sems[4:]
        wid = lax.axis_index("s") * num_cores + lax.axis_index("c")
        col0 = lax.mul(wid, BW)

        pltpu.sync_copy(idx_hbm.at[:, pl.ds(col0, BW)], idx_v)

        iota = lax.iota(jnp.int32, L)

        def fire_gather(h, q, s):
            for r in range(CH // L):
                sl = pl.ds(CH * q + r * L, L)
                phs[s, pl.ds(r * L, L)] = lax.shift_right_logical(
                    idx_v[h, sl], 1
                )
            pltpu.async_copy(table_hbm.at[phs.at[s]], bufs.at[s], gsem[s])

        def gather_wait(s):
            pltpu.make_async_copy(
                table_hbm.at[phs.at[s]], bufs.at[s], gsem[s]
            ).wait()

        def select(h, q, s, o):
            buf = bufs.at[s]
            ob = obs.at[o]

            @functools.partial(plsc.parallel_loop, 0, CH // L)
            def grp(r):
                jv = lax.add(lax.mul(r, L), iota)
                tokv = idx_v[
                    h, pl.ds(lax.add(lax.mul(CH, q), lax.mul(r, L)), L)
                ]
                hb = lax.mul(lax.bitwise_and(tokv, 1), D)
                for dblk in range(D // L):
                    vs = [
                        plsc.load_gather(buf, [jv, lax.add(hb, dblk * L + k)])
                        for k in range(L)
                    ]
                    for k in range(L):
                        ob[dblk * L + k, pl.ds(lax.mul(r, L), L)] = vs[k]

        def out_slice(h, q):
            return out_hbm.at[
                pl.ds(lax.mul(h, D), D), pl.ds(col0 + CH * q, CH)
            ]

        def wb_fire(h, q, o):
            pltpu.async_copy(obs.at[o], out_slice(h, q), wsem[o])

        def wb_wait(h, q, o):
            pltpu.make_async_copy(obs.at[o], out_slice(h, q), wsem[o]).wait()

        def step(h, q, first, last):
            o = q % 2
            if not last:
                if q < 2:
                    fire_gather(h, q + 2, q + 2)
                else:
                    fire_gather(h + 1, q - 2, q - 2)
            gather_wait(q)
            if not first:
                if q < 2:
                    wb_wait(h - 1, q + 2, o)
                else:
                    wb_wait(h, q - 2, o)
            select(h, q, q, o)
            wb_fire(h, q, o)

        fire_gather(0, 0, 0)
        fire_gather(0, 1, 1)
        step(0, 0, first=True, last=False)
        step(0, 1, first=True, last=False)
        step(0, 2, first=False, last=False)
        step(0, 3, first=False, last=False)

        def hbody(h, carry):
            for q in range(4):
                step(h, q, first=False, last=False)
            return carry

        lax.fori_loop(1, H - 1, hbody, 0)

        step(H - 1, 0, first=False, last=False)
        step(H - 1, 1, first=False, last=False)
        step(H - 1, 2, first=False, last=True)
        step(H - 1, 3, first=False, last=True)
        wb_wait(H - 1, 2, 0)
        wb_wait(H - 1, 3, 1)

    return body


def kernel(token_ids, embeddings):
    B, H = token_ids.shape
    V, D = embeddings.shape
    info = plsc.get_sparse_core_info()
    ids_t = token_ids.T.astype(jnp.int32)
    table2 = embeddings.reshape(V // 2, 2 * D)
    out_t = _make_kernel(B, H, V // 2, D, info.num_cores, info.num_subcores)(
        ids_t, table2
    )
    return out_t.reshape(H, D, B).transpose(2, 0, 1)
